# baseline (speedup 1.0000x reference)
import jax
import jax.numpy as jnp
from jax import lax
from jax.experimental import pallas as pl
from jax.experimental.pallas import tpu as pltpu
from jax.experimental.pallas import tpu_sc as plsc

N_EXPERTS = 64
K_TOP = 8
HIDDEN = 4096
N_ROWS = 16384
BM = 1024

_info = plsc.get_sparse_core_info()
_NC, _NS, _NL = _info.num_cores, _info.num_subcores, _info.num_lanes
_NW = _NC * _NS
_ROWS_W = N_ROWS // _NW


def _gate_body(x_ref, wt_ref, probs_ref):
    x = x_ref[...]
    wt = wt_ref[...]
    logits = lax.dot_general(
        x, wt, (((1,), (0,)), ((), ())),
        preferred_element_type=jnp.float32,
        precision=lax.Precision.DEFAULT,
    )
    m = jnp.max(logits, axis=1, keepdims=True)
    e = jnp.exp(logits - m)
    probs_ref[...] = e / jnp.sum(e, axis=1, keepdims=True)


def _gate_probs(x, wt):
    return pl.pallas_call(
        _gate_body,
        grid=(N_ROWS // BM,),
        in_specs=[
            pl.BlockSpec((BM, HIDDEN), lambda i: (i, 0)),
            pl.BlockSpec((HIDDEN, N_EXPERTS), lambda i: (0, 0)),
        ],
        out_specs=pl.BlockSpec((BM, N_EXPERTS), lambda i: (i, 0)),
        out_shape=jax.ShapeDtypeStruct((N_ROWS, N_EXPERTS), jnp.float32),
        compiler_params=pltpu.CompilerParams(
            dimension_semantics=("arbitrary",),
        ),
    )(x, wt)


def _merge_top(k0, v0, k1, v1):
    rk = lax.rev(k1, (0,))
    rv = lax.rev(v1, (0,))
    m = k0 >= rk
    ck = jnp.where(m, k0, rk)
    cv = jnp.where(m, v0, rv)
    return plsc.sort_key_val(ck, cv, descending=True)


def _topk_body(probs_hbm, idx_hbm, tw_hbm, pbuf, ibuf, wbuf):
    wid = lax.axis_index("s") * _NC + lax.axis_index("c")
    base = wid * _ROWS_W
    pltpu.sync_copy(probs_hbm.at[pl.ds(base * N_EXPERTS, _ROWS_W * N_EXPERTS)], pbuf)

    lane = lax.iota(jnp.int32, _NL)
    sel8 = lane < K_TOP
    idx_consts = [lane + 16 * j for j in range(4)]

    def row(r, carry):
        ks, vs = [], []
        for j in range(4):
            k = pbuf[pl.ds(r * N_EXPERTS + 16 * j, 16)]
            kk, vv = plsc.sort_key_val(k, idx_consts[j], descending=True)
            ks.append(kk)
            vs.append(vv)
        k01, v01 = _merge_top(ks[0], vs[0], ks[1], vs[1])
        k23, v23 = _merge_top(ks[2], vs[2], ks[3], vs[3])
        kf, vf = _merge_top(k01, v01, k23, v23)
        s = jnp.sum(jnp.where(sel8, kf, 0.0))
        tw = kf / s
        pos = r * K_TOP + lane
        plsc.store_scatter(wbuf, [pos], tw, mask=sel8)
        plsc.store_scatter(ibuf, [pos], vf, mask=sel8)
        return carry

    lax.fori_loop(0, _ROWS_W, row, 0)
    pltpu.sync_copy(ibuf, idx_hbm.at[pl.ds(base * K_TOP, _ROWS_W * K_TOP)])
    pltpu.sync_copy(wbuf, tw_hbm.at[pl.ds(base * K_TOP, _ROWS_W * K_TOP)])


def _topk_sc(probs_flat):
    mesh = plsc.VectorSubcoreMesh(core_axis_name="c", subcore_axis_name="s")
    f = pl.kernel(
        _topk_body,
        out_type=[
            jax.ShapeDtypeStruct((N_ROWS * K_TOP,), jnp.int32),
            jax.ShapeDtypeStruct((N_ROWS * K_TOP,), jnp.float32),
        ],
        mesh=mesh,
        scratch_types=[
            pltpu.VMEM((_ROWS_W * N_EXPERTS,), jnp.float32),
            pltpu.VMEM((_ROWS_W * K_TOP,), jnp.int32),
            pltpu.VMEM((_ROWS_W * K_TOP,), jnp.float32),
        ],
        compiler_params=pltpu.CompilerParams(needs_layout_passes=False),
    )
    return f(probs_flat)


def kernel(x, W):
    wt = W.T
    probs = _gate_probs(x, wt)
    idx_flat, tw_flat = _topk_sc(probs.reshape(-1))
    return (
        idx_flat.reshape(N_ROWS, K_TOP),
        tw_flat.reshape(N_ROWS, K_TOP),
        probs,
    )


# speedup vs baseline: 1.1492x; 1.1492x over previous
import jax
import jax.numpy as jnp
from jax import lax
from jax.experimental import pallas as pl
from jax.experimental.pallas import tpu as pltpu
from jax.experimental.pallas import tpu_sc as plsc

N_EXPERTS = 64
K_TOP = 8
HIDDEN = 4096
N_ROWS = 16384
BM = 1024

_info = plsc.get_sparse_core_info()
_NC, _NS, _NL = _info.num_cores, _info.num_subcores, _info.num_lanes
_NW = _NC * _NS
_ROWS_W = N_ROWS // _NW


def _gate_body(x_ref, wt_ref, probs_ref):
    x = x_ref[...]
    wt = wt_ref[...]
    logits = lax.dot_general(
        x, wt, (((1,), (0,)), ((), ())),
        preferred_element_type=jnp.float32,
        precision=lax.Precision.DEFAULT,
    )
    m = jnp.max(logits, axis=1, keepdims=True)
    e = jnp.exp(logits - m)
    probs_ref[...] = e / jnp.sum(e, axis=1, keepdims=True)


def _gate_probs(x, wt):
    return pl.pallas_call(
        _gate_body,
        grid=(N_ROWS // BM,),
        in_specs=[
            pl.BlockSpec((BM, HIDDEN), lambda i: (i, 0)),
            pl.BlockSpec((HIDDEN, N_EXPERTS), lambda i: (0, 0)),
        ],
        out_specs=pl.BlockSpec((BM, N_EXPERTS), lambda i: (i, 0)),
        out_shape=jax.ShapeDtypeStruct((N_ROWS, N_EXPERTS), jnp.float32),
        compiler_params=pltpu.CompilerParams(
            dimension_semantics=("arbitrary",),
        ),
    )(x, wt)


def _merge_top(k0, v0, k1, v1):
    rk = lax.rev(k1, (0,))
    rv = lax.rev(v1, (0,))
    m = k0 >= rk
    ck = jnp.where(m, k0, rk)
    cv = jnp.where(m, v0, rv)
    return plsc.sort_key_val(ck, cv, descending=True)


def _topk_body(probs_hbm, idx_hbm, tw_hbm, pbuf, ibuf, wbuf):
    wid = lax.axis_index("s") * _NC + lax.axis_index("c")
    base = wid * _ROWS_W
    pltpu.sync_copy(probs_hbm.at[pl.ds(base * N_EXPERTS, _ROWS_W * N_EXPERTS)], pbuf)

    lane = lax.iota(jnp.int32, _NL)
    sel8 = lane < K_TOP
    idx_consts = [lane + 16 * j for j in range(4)]

    @plsc.parallel_loop(0, _ROWS_W, unroll=4)
    def row(r):
        ks, vs = [], []
        for j in range(4):
            k = pbuf[pl.ds(r * N_EXPERTS + 16 * j, 16)]
            kk, vv = plsc.sort_key_val(k, idx_consts[j], descending=True)
            ks.append(kk)
            vs.append(vv)
        k01, v01 = _merge_top(ks[0], vs[0], ks[1], vs[1])
        k23, v23 = _merge_top(ks[2], vs[2], ks[3], vs[3])
        kf, vf = _merge_top(k01, v01, k23, v23)
        s = jnp.sum(jnp.where(sel8, kf, 0.0))
        tw = kf / s
        pos = r * K_TOP + lane
        plsc.store_scatter(wbuf, [pos], tw, mask=sel8)
        plsc.store_scatter(ibuf, [pos], vf, mask=sel8)
    pltpu.sync_copy(ibuf, idx_hbm.at[pl.ds(base * K_TOP, _ROWS_W * K_TOP)])
    pltpu.sync_copy(wbuf, tw_hbm.at[pl.ds(base * K_TOP, _ROWS_W * K_TOP)])


def _topk_sc(probs_flat):
    mesh = plsc.VectorSubcoreMesh(core_axis_name="c", subcore_axis_name="s")
    f = pl.kernel(
        _topk_body,
        out_type=[
            jax.ShapeDtypeStruct((N_ROWS * K_TOP,), jnp.int32),
            jax.ShapeDtypeStruct((N_ROWS * K_TOP,), jnp.float32),
        ],
        mesh=mesh,
        scratch_types=[
            pltpu.VMEM((_ROWS_W * N_EXPERTS,), jnp.float32),
            pltpu.VMEM((_ROWS_W * K_TOP,), jnp.int32),
            pltpu.VMEM((_ROWS_W * K_TOP,), jnp.float32),
        ],
        compiler_params=pltpu.CompilerParams(needs_layout_passes=False),
    )
    return f(probs_flat)


def kernel(x, W):
    wt = W.T
    probs = _gate_probs(x, wt)
    idx_flat, tw_flat = _topk_sc(probs.reshape(-1))
    return (
        idx_flat.reshape(N_ROWS, K_TOP),
        tw_flat.reshape(N_ROWS, K_TOP),
        probs,
    )
